# Initial kernel scaffold; baseline (speedup 1.0000x reference)
import jax
import jax.numpy as jnp
from jax.experimental import pallas as pl


def kernel(x, edge_index, W, b):
    raise NotImplementedError("write your pallas kernel here")


# speedup vs baseline: 17.0266x; 17.0266x over previous
import functools

import jax
import jax.numpy as jnp
from jax import lax
from jax.experimental import pallas as pl
from jax.experimental.pallas import tpu as pltpu
from jax.experimental.pallas import tpu_sc as plsc

N = 10000
E = 320000
D = 128

NC = 2
NS = 16
NW = NC * NS
EPT = E // NW
K = 80
NCHUNK = EPT // K
NP = 10112
RPT = NP // NS
DW = 16

@functools.lru_cache(maxsize=None)
def _mesh():
    return plsc.VectorSubcoreMesh(core_axis_name="c", subcore_axis_name="s",
                                  num_cores=NC, num_subcores=NS)


def _zero_rows(ref, nrows, ncol16):
    def body(i, _):
        for j in range(ncol16):
            ref[i, pl.ds(j * 16, 16)] = jnp.zeros((16,), jnp.float32)
        return 0
    lax.fori_loop(0, nrows, body, 0)


def _fill_span(buf, dst, d_base, nrows):
    full, rem = nrows // K, nrows % K
    for j in range(full):
        pltpu.sync_copy(buf, dst.at[pl.ds(d_base + j * K, K)])
    if rem:
        pltpu.sync_copy(buf.at[pl.ds(0, rem)],
                        dst.at[pl.ds(d_base + full * K, rem)])


def _deg_body(dst_hbm, out_hbm, buf_v, idx_v, deg_sh):
    c = lax.axis_index("c")
    s = lax.axis_index("s")
    t = s * NC + c
    r0 = s * RPT
    _zero_rows(buf_v, K, DW // 16)
    _fill_span(buf_v, deg_sh, r0, RPT)

    def ones_row(i, _):
        buf_v[i, :] = jnp.ones((DW,), jnp.float32)
        return 0
    lax.fori_loop(0, K, ones_row, 0)
    plsc.subcore_barrier()

    def chunk(ci, _):
        base = pl.multiple_of(t * EPT + ci * K, 8)
        pltpu.sync_copy(dst_hbm.at[pl.ds(base, K)], idx_v)
        pltpu.sync_copy(buf_v, deg_sh.at[idx_v], add=True)
        return 0
    lax.fori_loop(0, NCHUNK, chunk, 0)
    plsc.subcore_barrier()

    o0 = c * NP + r0
    full, rem = RPT // K, RPT % K
    for j in range(full):
        pltpu.sync_copy(deg_sh.at[pl.ds(r0 + j * K, K)], buf_v)
        pltpu.sync_copy(buf_v, out_hbm.at[pl.ds(o0 + j * K, K)])
    if rem:
        pltpu.sync_copy(deg_sh.at[pl.ds(r0 + full * K, rem)],
                        buf_v.at[pl.ds(0, rem)])
        pltpu.sync_copy(buf_v.at[pl.ds(0, rem)],
                        out_hbm.at[pl.ds(o0 + full * K, rem)])


@functools.lru_cache(maxsize=None)
def _sc_deg():
    return pl.kernel(
        _deg_body,
        out_type=jax.ShapeDtypeStruct((NC * NP, DW), jnp.float32),
        mesh=_mesh(),
        scratch_types=[
            pltpu.VMEM((K, DW), jnp.float32),
            pltpu.VMEM((K,), jnp.int32),
            pltpu.VMEM_SHARED((NP, DW), jnp.float32),
        ],
    )


def _agg_body(y_hbm, src_hbm, dst_hbm, out_hbm, rows_v, src_v, dst_v,
              acc_sh, sem):
    c = lax.axis_index("c")
    s = lax.axis_index("s")
    t = s * NC + c
    r0 = s * RPT
    _zero_rows(rows_v, K, D // 16)
    _fill_span(rows_v, acc_sh, r0, RPT)
    plsc.subcore_barrier()

    def chunk(ci, _):
        base = pl.multiple_of(t * EPT + ci * K, 8)
        pltpu.sync_copy(src_hbm.at[pl.ds(base, K)], src_v)
        pltpu.sync_copy(dst_hbm.at[pl.ds(base, K)], dst_v)
        pltpu.async_copy(y_hbm.at[src_v], rows_v, sem).wait()
        pltpu.sync_copy(rows_v, acc_sh.at[dst_v], add=True)
        return 0
    lax.fori_loop(0, NCHUNK, chunk, 0)
    plsc.subcore_barrier()

    o0 = c * NP + r0
    full, rem = RPT // K, RPT % K
    for j in range(full):
        pltpu.sync_copy(acc_sh.at[pl.ds(r0 + j * K, K)], rows_v)
        pltpu.sync_copy(rows_v, out_hbm.at[pl.ds(o0 + j * K, K)])
    if rem:
        pltpu.sync_copy(acc_sh.at[pl.ds(r0 + full * K, rem)],
                        rows_v.at[pl.ds(0, rem)])
        pltpu.sync_copy(rows_v.at[pl.ds(0, rem)],
                        out_hbm.at[pl.ds(o0 + full * K, rem)])


@functools.lru_cache(maxsize=None)
def _sc_agg():
    return pl.kernel(
        _agg_body,
        out_type=jax.ShapeDtypeStruct((NC * NP, D), jnp.float32),
        mesh=_mesh(),
        scratch_types=[
            pltpu.VMEM((K, D), jnp.float32),
            pltpu.VMEM((K,), jnp.int32),
            pltpu.VMEM((K,), jnp.int32),
            pltpu.VMEM_SHARED((NP, D), jnp.float32),
            pltpu.SemaphoreType.DMA,
        ],
    )


_RB = 1000


def _mm_body(x_ref, w_ref, o_ref):
    o_ref[...] = jnp.dot(x_ref[...], w_ref[...],
                         preferred_element_type=jnp.float32)


def _tc_matmul(x, W):
    return pl.pallas_call(
        _mm_body,
        grid=(N // _RB,),
        in_specs=[
            pl.BlockSpec((_RB, D), lambda i: (i, 0)),
            pl.BlockSpec((D, D), lambda i: (0, 0)),
        ],
        out_specs=pl.BlockSpec((_RB, D), lambda i: (i, 0)),
        out_shape=jax.ShapeDtypeStruct((N, D), jnp.float32),
    )(x, W)


def _scale_body(xw_ref, da_ref, db_ref, y_ref):
    deg = da_ref[:, 0:1] + db_ref[:, 0:1] + 1.0
    y_ref[...] = xw_ref[...] * lax.rsqrt(deg)


def _tc_scale(xw, dega, degb):
    return pl.pallas_call(
        _scale_body,
        grid=(N // _RB,),
        in_specs=[
            pl.BlockSpec((_RB, D), lambda i: (i, 0)),
            pl.BlockSpec((_RB, DW), lambda i: (i, 0)),
            pl.BlockSpec((_RB, DW), lambda i: (i, 0)),
        ],
        out_specs=pl.BlockSpec((_RB, D), lambda i: (i, 0)),
        out_shape=jax.ShapeDtypeStruct((N, D), jnp.float32),
    )(xw, dega, degb)


def _final_body(aa_ref, ab_ref, y_ref, da_ref, db_ref, b_ref, o_ref):
    deg = da_ref[:, 0:1] + db_ref[:, 0:1] + 1.0
    dinv = lax.rsqrt(deg)
    acc = aa_ref[...] + ab_ref[...] + y_ref[...]
    o_ref[...] = jnp.maximum(dinv * acc + b_ref[...], 0.0)


def _tc_final(agga, aggb, y, dega, degb, b):
    return pl.pallas_call(
        _final_body,
        grid=(N // _RB,),
        in_specs=[
            pl.BlockSpec((_RB, D), lambda i: (i, 0)),
            pl.BlockSpec((_RB, D), lambda i: (i, 0)),
            pl.BlockSpec((_RB, D), lambda i: (i, 0)),
            pl.BlockSpec((_RB, DW), lambda i: (i, 0)),
            pl.BlockSpec((_RB, DW), lambda i: (i, 0)),
            pl.BlockSpec((1, D), lambda i: (0, 0)),
        ],
        out_specs=pl.BlockSpec((_RB, D), lambda i: (i, 0)),
        out_shape=jax.ShapeDtypeStruct((N, D), jnp.float32),
    )(agga, aggb, y, dega, degb, b)


def kernel(x, edge_index, W, b):
    src = edge_index[0].astype(jnp.int32)
    dst = edge_index[1].astype(jnp.int32)

    degp = _sc_deg()(dst)
    dega, degb = degp[:N], degp[NP:NP + N]
    xw = _tc_matmul(x, W)
    y = _tc_scale(xw, dega, degb)
    aggp = _sc_agg()(y, src, dst)
    return _tc_final(aggp[:N], aggp[NP:NP + N], y, dega, degb,
                     b.reshape(1, D).astype(jnp.float32))


# speedup vs baseline: 39.9015x; 2.3435x over previous
import functools

import jax
import jax.numpy as jnp
from jax import lax
from jax.experimental import pallas as pl
from jax.experimental.pallas import tpu as pltpu
from jax.experimental.pallas import tpu_sc as plsc

N = 10000
E = 320000
D = 128

NC = 2
NS = 16
NW = NC * NS
K = 80
NCHUNK = 125
EPT = E // NW
NP = 10112
RPT = NP // NS
DW = 16

@functools.lru_cache(maxsize=None)
def _mesh():
    return plsc.VectorSubcoreMesh(core_axis_name="c", subcore_axis_name="s",
                                  num_cores=NC, num_subcores=NS)


def _zero_rows(ref, nrows, ncol16):
    def body(i, _):
        for j in range(ncol16):
            ref[i, pl.ds(j * 16, 16)] = jnp.zeros((16,), jnp.float32)
        return 0
    lax.fori_loop(0, nrows, body, 0)


def _fill_span(buf, dst, d_base, nrows):
    full, rem = nrows // K, nrows % K
    for j in range(full):
        pltpu.sync_copy(buf, dst.at[pl.ds(d_base + j * K, K)])
    if rem:
        pltpu.sync_copy(buf.at[pl.ds(0, rem)],
                        dst.at[pl.ds(d_base + full * K, rem)])


R = 5
RB = 4


def _load_idx2d(hbm_1d, idx2d_v, t, sem):
    def fire(ci, u):
        base = pl.multiple_of(t * EPT + ci * K, 8)
        pltpu.async_copy(hbm_1d.at[pl.ds(base, K)], idx2d_v.at[ci],
                         sem.at[u])

    def drain(u):
        pltpu.make_async_copy(hbm_1d.at[pl.ds(0, K)], idx2d_v.at[0],
                              sem.at[u]).wait()

    for u in range(R):
        fire(u, u)

    def step(gi, _):
        for u in range(R):
            drain(u)
            fire(gi * R + u, u)
        return 0
    lax.fori_loop(1, NCHUNK // R, step, 0)
    for u in range(R):
        drain(u)


def _deg_body(dst_hbm, out_hbm, buf_v, idx_v, deg_sh, sem_i, sem_s):
    c = lax.axis_index("c")
    s = lax.axis_index("s")
    t = s * NC + c
    r0 = s * RPT
    _zero_rows(buf_v, K, DW // 16)
    _fill_span(buf_v, deg_sh, r0, RPT)

    def ones_row(i, _):
        buf_v[i, :] = jnp.ones((DW,), jnp.float32)
        return 0
    lax.fori_loop(0, K, ones_row, 0)
    _load_idx2d(dst_hbm, idx_v, t, sem_i)
    plsc.subcore_barrier()

    def start_s(ci, u):
        pltpu.async_copy(buf_v, deg_sh.at[idx_v.at[ci]], sem_s.at[u],
                         add=True)

    def wait_s(u):
        pltpu.make_async_copy(buf_v, deg_sh.at[idx_v.at[0]],
                              sem_s.at[u]).wait()

    for u in range(R):
        start_s(u, u)

    def step(gi, _):
        for u in range(R):
            wait_s(u)
            start_s(gi * R + u, u)
        return 0
    lax.fori_loop(1, NCHUNK // R, step, 0)
    for u in range(R):
        wait_s(u)
    plsc.subcore_barrier()

    o0 = c * NP + r0
    full, rem = RPT // K, RPT % K
    for j in range(full):
        pltpu.sync_copy(deg_sh.at[pl.ds(r0 + j * K, K)], buf_v)
        pltpu.sync_copy(buf_v, out_hbm.at[pl.ds(o0 + j * K, K)])
    if rem:
        pltpu.sync_copy(deg_sh.at[pl.ds(r0 + full * K, rem)],
                        buf_v.at[pl.ds(0, rem)])
        pltpu.sync_copy(buf_v.at[pl.ds(0, rem)],
                        out_hbm.at[pl.ds(o0 + full * K, rem)])


@functools.lru_cache(maxsize=None)
def _sc_deg():
    return pl.kernel(
        _deg_body,
        out_type=jax.ShapeDtypeStruct((NC * NP, DW), jnp.float32),
        mesh=_mesh(),
        scratch_types=[
            pltpu.VMEM((K, DW), jnp.float32),
            pltpu.VMEM((NCHUNK, K), jnp.int32),
            pltpu.VMEM_SHARED((NP, DW), jnp.float32),
            pltpu.SemaphoreType.DMA((R,)),
            pltpu.SemaphoreType.DMA((R,)),
        ],
    )


def _agg_body(y_hbm, src_hbm, dst_hbm, out_hbm, bufs_v, srcr_v, dstr_v,
              acc_sh, sem_is, sem_id, sem_g, sem_s):
    c = lax.axis_index("c")
    s = lax.axis_index("s")
    t = s * NC + c
    r0 = s * RPT
    _zero_rows(bufs_v.at[0], K, D // 16)
    _fill_span(bufs_v.at[0], acc_sh, r0, RPT)
    plsc.subcore_barrier()

    def start_il(ci, u):
        base = pl.multiple_of(t * EPT, 8) + ci * K
        pltpu.async_copy(src_hbm.at[pl.ds(base, K)], srcr_v.at[u],
                         sem_is.at[u])
        pltpu.async_copy(dst_hbm.at[pl.ds(base, K)], dstr_v.at[u],
                         sem_id.at[u])

    def wait_il(u):
        pltpu.make_async_copy(src_hbm.at[pl.ds(0, K)], srcr_v.at[u],
                              sem_is.at[u]).wait()
        pltpu.make_async_copy(dst_hbm.at[pl.ds(0, K)], dstr_v.at[u],
                              sem_id.at[u]).wait()

    def start_g(u):
        pltpu.async_copy(y_hbm.at[srcr_v.at[u]], bufs_v.at[u], sem_g.at[u])

    def wait_g(u):
        pltpu.make_async_copy(y_hbm.at[srcr_v.at[u]], bufs_v.at[u],
                              sem_g.at[u]).wait()

    def start_s(u):
        pltpu.async_copy(bufs_v.at[u], acc_sh.at[dstr_v.at[u]],
                         sem_s.at[u], add=True)

    def wait_s(u):
        pltpu.make_async_copy(bufs_v.at[u], acc_sh.at[dstr_v.at[u]],
                              sem_s.at[u]).wait()

    LAST = NCHUNK - 1
    start_il(0, 0)
    start_il(1, 1)
    start_il(2, 2)
    wait_il(0)
    start_g(0)
    wait_il(1)
    start_g(1)
    for i in range(RB):
        if i >= 1:
            wait_s(i - 1)
        start_il(i + 3, (i + 3) % RB)
        wait_il((i + 2) % RB)
        start_g((i + 2) % RB)
        wait_g(i)
        start_s(i)

    def step(gi, _):
        for u in range(RB):
            i = gi * RB + u
            wait_s((u + 3) % RB)
            start_il(jnp.minimum(i + 3, LAST), (u + 3) % RB)
            wait_il((u + 2) % RB)
            start_g((u + 2) % RB)
            wait_g(u)
            start_s(u)
        return 0
    lax.fori_loop(1, NCHUNK // RB, step, 0)

    wait_s(3)
    wait_g(0)
    start_s(0)
    wait_s(0)
    wait_g(1)
    wait_il(2)
    plsc.subcore_barrier()

    o0 = c * NP + r0
    full, rem = RPT // K, RPT % K
    for j in range(full):
        pltpu.sync_copy(acc_sh.at[pl.ds(r0 + j * K, K)], bufs_v.at[0])
        pltpu.sync_copy(bufs_v.at[0], out_hbm.at[pl.ds(o0 + j * K, K)])
    if rem:
        pltpu.sync_copy(acc_sh.at[pl.ds(r0 + full * K, rem)],
                        bufs_v.at[0].at[pl.ds(0, rem)])
        pltpu.sync_copy(bufs_v.at[0].at[pl.ds(0, rem)],
                        out_hbm.at[pl.ds(o0 + full * K, rem)])


@functools.lru_cache(maxsize=None)
def _sc_agg():
    return pl.kernel(
        _agg_body,
        out_type=jax.ShapeDtypeStruct((NC * NP, D), jnp.float32),
        mesh=_mesh(),
        scratch_types=[
            pltpu.VMEM((RB, K, D), jnp.float32),
            pltpu.VMEM((RB, K), jnp.int32),
            pltpu.VMEM((RB, K), jnp.int32),
            pltpu.VMEM_SHARED((NP, D), jnp.float32),
            pltpu.SemaphoreType.DMA((RB,)),
            pltpu.SemaphoreType.DMA((RB,)),
            pltpu.SemaphoreType.DMA((RB,)),
            pltpu.SemaphoreType.DMA((RB,)),
        ],
    )


_RB = 1000


def _mm_body(x_ref, w_ref, o_ref):
    o_ref[...] = jnp.dot(x_ref[...], w_ref[...],
                         preferred_element_type=jnp.float32)


def _tc_matmul(x, W):
    return pl.pallas_call(
        _mm_body,
        grid=(N // _RB,),
        in_specs=[
            pl.BlockSpec((_RB, D), lambda i: (i, 0)),
            pl.BlockSpec((D, D), lambda i: (0, 0)),
        ],
        out_specs=pl.BlockSpec((_RB, D), lambda i: (i, 0)),
        out_shape=jax.ShapeDtypeStruct((N, D), jnp.float32),
    )(x, W)


def _scale_body(xw_ref, da_ref, db_ref, y_ref):
    deg = da_ref[:, 0:1] + db_ref[:, 0:1] + 1.0
    y_ref[...] = xw_ref[...] * lax.rsqrt(deg)


def _tc_scale(xw, dega, degb):
    return pl.pallas_call(
        _scale_body,
        grid=(N // _RB,),
        in_specs=[
            pl.BlockSpec((_RB, D), lambda i: (i, 0)),
            pl.BlockSpec((_RB, DW), lambda i: (i, 0)),
            pl.BlockSpec((_RB, DW), lambda i: (i, 0)),
        ],
        out_specs=pl.BlockSpec((_RB, D), lambda i: (i, 0)),
        out_shape=jax.ShapeDtypeStruct((N, D), jnp.float32),
    )(xw, dega, degb)


def _final_body(aa_ref, ab_ref, y_ref, da_ref, db_ref, b_ref, o_ref):
    deg = da_ref[:, 0:1] + db_ref[:, 0:1] + 1.0
    dinv = lax.rsqrt(deg)
    acc = aa_ref[...] + ab_ref[...] + y_ref[...]
    o_ref[...] = jnp.maximum(dinv * acc + b_ref[...], 0.0)


def _tc_final(agga, aggb, y, dega, degb, b):
    return pl.pallas_call(
        _final_body,
        grid=(N // _RB,),
        in_specs=[
            pl.BlockSpec((_RB, D), lambda i: (i, 0)),
            pl.BlockSpec((_RB, D), lambda i: (i, 0)),
            pl.BlockSpec((_RB, D), lambda i: (i, 0)),
            pl.BlockSpec((_RB, DW), lambda i: (i, 0)),
            pl.BlockSpec((_RB, DW), lambda i: (i, 0)),
            pl.BlockSpec((1, D), lambda i: (0, 0)),
        ],
        out_specs=pl.BlockSpec((_RB, D), lambda i: (i, 0)),
        out_shape=jax.ShapeDtypeStruct((N, D), jnp.float32),
    )(agga, aggb, y, dega, degb, b)


def kernel(x, edge_index, W, b):
    src = edge_index[0].astype(jnp.int32)
    dst = edge_index[1].astype(jnp.int32)

    degp = _sc_deg()(dst)
    dega, degb = degp[:N], degp[NP:NP + N]
    xw = _tc_matmul(x, W)
    y = _tc_scale(xw, dega, degb)
    aggp = _sc_agg()(y, src, dst)
    return _tc_final(aggp[:N], aggp[NP:NP + N], y, dega, degb,
                     b.reshape(1, D).astype(jnp.float32))


# speedup vs baseline: 42.9423x; 1.0762x over previous
import functools

import jax
import jax.numpy as jnp
from jax import lax
from jax.experimental import pallas as pl
from jax.experimental.pallas import tpu as pltpu
from jax.experimental.pallas import tpu_sc as plsc

N = 10000
E = 320000
D = 128

NC = 2
NS = 16
NW = NC * NS
K = 80
NCHUNK = 125
EPT = E // NW
NP = 10112
RPT = NP // NS
DW = 16

@functools.lru_cache(maxsize=None)
def _mesh():
    return plsc.VectorSubcoreMesh(core_axis_name="c", subcore_axis_name="s",
                                  num_cores=NC, num_subcores=NS)


def _zero_rows(ref, nrows, ncol16):
    def body(i, _):
        for j in range(ncol16):
            ref[i, pl.ds(j * 16, 16)] = jnp.zeros((16,), jnp.float32)
        return 0
    lax.fori_loop(0, nrows, body, 0)


def _fill_span(buf, dst, d_base, nrows):
    full, rem = nrows // K, nrows % K
    for j in range(full):
        pltpu.sync_copy(buf, dst.at[pl.ds(d_base + j * K, K)])
    if rem:
        pltpu.sync_copy(buf.at[pl.ds(0, rem)],
                        dst.at[pl.ds(d_base + full * K, rem)])


R = 5
RD = 5
RB = 4


def _load_idx2d(hbm_1d, idx2d_v, t, sem):
    def fire(ci, u):
        base = pl.multiple_of(t * EPT + ci * K, 8)
        pltpu.async_copy(hbm_1d.at[pl.ds(base, K)], idx2d_v.at[ci],
                         sem.at[u])

    def drain(u):
        pltpu.make_async_copy(hbm_1d.at[pl.ds(0, K)], idx2d_v.at[0],
                              sem.at[u]).wait()

    for u in range(R):
        fire(u, u)

    def step(gi, _):
        for u in range(R):
            drain(u)
            fire(gi * R + u, u)
        return 0
    lax.fori_loop(1, NCHUNK // R, step, 0)
    for u in range(R):
        drain(u)


def _deg_body(dst_hbm, out_hbm, buf_v, idx_v, deg_sh, sem_i, sem_s):
    c = lax.axis_index("c")
    s = lax.axis_index("s")
    t = s * NC + c
    r0 = s * RPT
    _zero_rows(buf_v, K, DW // 16)
    _fill_span(buf_v, deg_sh, r0, RPT)

    def ones_row(i, _):
        buf_v[i, :] = jnp.ones((DW,), jnp.float32)
        return 0
    lax.fori_loop(0, K, ones_row, 0)
    _load_idx2d(dst_hbm, idx_v, t, sem_i)
    plsc.subcore_barrier()

    def start_s(ci, u):
        pltpu.async_copy(buf_v, deg_sh.at[idx_v.at[ci]], sem_s.at[u],
                         add=True)

    def wait_s(u):
        pltpu.make_async_copy(buf_v, deg_sh.at[idx_v.at[0]],
                              sem_s.at[u]).wait()

    for u in range(RD):
        start_s(u, u)

    def step(gi, _):
        for u in range(RD):
            wait_s(u)
            start_s(gi * RD + u, u)
        return 0
    lax.fori_loop(1, NCHUNK // RD, step, 0)
    for u in range(RD):
        wait_s(u)
    plsc.subcore_barrier()

    o0 = c * NP + r0
    full, rem = RPT // K, RPT % K
    for j in range(full):
        pltpu.sync_copy(deg_sh.at[pl.ds(r0 + j * K, K)], buf_v)
        pltpu.sync_copy(buf_v, out_hbm.at[pl.ds(o0 + j * K, K)])
    if rem:
        pltpu.sync_copy(deg_sh.at[pl.ds(r0 + full * K, rem)],
                        buf_v.at[pl.ds(0, rem)])
        pltpu.sync_copy(buf_v.at[pl.ds(0, rem)],
                        out_hbm.at[pl.ds(o0 + full * K, rem)])


@functools.lru_cache(maxsize=None)
def _sc_deg():
    return pl.kernel(
        _deg_body,
        out_type=jax.ShapeDtypeStruct((NC * NP, DW), jnp.float32),
        mesh=_mesh(),
        scratch_types=[
            pltpu.VMEM((K, DW), jnp.float32),
            pltpu.VMEM((NCHUNK, K), jnp.int32),
            pltpu.VMEM_SHARED((NP, DW), jnp.float32),
            pltpu.SemaphoreType.DMA((R,)),
            pltpu.SemaphoreType.DMA((RD,)),
        ],
    )


def _agg_body(y_hbm, src_hbm, dst_hbm, out_hbm, bufs_v, srcr_v, dstr_v,
              acc_sh, sem_is, sem_id, sem_g, sem_s):
    c = lax.axis_index("c")
    s = lax.axis_index("s")
    t = s * NC + c
    r0 = s * RPT
    _zero_rows(bufs_v.at[0], K, D // 16)
    _fill_span(bufs_v.at[0], acc_sh, r0, RPT)
    plsc.subcore_barrier()

    def start_il(ci, u):
        base = pl.multiple_of(t * EPT, 8) + ci * K
        pltpu.async_copy(src_hbm.at[pl.ds(base, K)], srcr_v.at[u],
                         sem_is.at[u])
        pltpu.async_copy(dst_hbm.at[pl.ds(base, K)], dstr_v.at[u],
                         sem_id.at[u])

    def wait_il(u):
        pltpu.make_async_copy(src_hbm.at[pl.ds(0, K)], srcr_v.at[u],
                              sem_is.at[u]).wait()
        pltpu.make_async_copy(dst_hbm.at[pl.ds(0, K)], dstr_v.at[u],
                              sem_id.at[u]).wait()

    def start_g(u):
        pltpu.async_copy(y_hbm.at[srcr_v.at[u]], bufs_v.at[u], sem_g.at[u])

    def wait_g(u):
        pltpu.make_async_copy(y_hbm.at[srcr_v.at[u]], bufs_v.at[u],
                              sem_g.at[u]).wait()

    def start_s(u):
        pltpu.async_copy(bufs_v.at[u], acc_sh.at[dstr_v.at[u]],
                         sem_s.at[u], add=True)

    def wait_s(u):
        pltpu.make_async_copy(bufs_v.at[u], acc_sh.at[dstr_v.at[u]],
                              sem_s.at[u]).wait()

    LAST = NCHUNK - 1
    start_il(0, 0)
    start_il(1, 1)
    start_il(2, 2)
    wait_il(0)
    start_g(0)
    wait_il(1)
    start_g(1)
    for i in range(RB):
        if i >= 1:
            wait_s(i - 1)
        start_il(i + 3, (i + 3) % RB)
        wait_il((i + 2) % RB)
        start_g((i + 2) % RB)
        wait_g(i)
        start_s(i)

    def step(gi, _):
        for u in range(RB):
            i = gi * RB + u
            wait_s((u + 3) % RB)
            start_il(jnp.minimum(i + 3, LAST), (u + 3) % RB)
            wait_il((u + 2) % RB)
            start_g((u + 2) % RB)
            wait_g(u)
            start_s(u)
        return 0
    lax.fori_loop(1, NCHUNK // RB, step, 0)

    wait_s(3)
    wait_g(0)
    start_s(0)
    wait_s(0)
    wait_g(1)
    wait_il(2)
    plsc.subcore_barrier()

    o0 = c * NP + r0
    full, rem = RPT // K, RPT % K
    for j in range(full):
        pltpu.sync_copy(acc_sh.at[pl.ds(r0 + j * K, K)], bufs_v.at[0])
        pltpu.sync_copy(bufs_v.at[0], out_hbm.at[pl.ds(o0 + j * K, K)])
    if rem:
        pltpu.sync_copy(acc_sh.at[pl.ds(r0 + full * K, rem)],
                        bufs_v.at[0].at[pl.ds(0, rem)])
        pltpu.sync_copy(bufs_v.at[0].at[pl.ds(0, rem)],
                        out_hbm.at[pl.ds(o0 + full * K, rem)])


@functools.lru_cache(maxsize=None)
def _sc_agg():
    return pl.kernel(
        _agg_body,
        out_type=jax.ShapeDtypeStruct((NC * NP, D), jnp.float32),
        mesh=_mesh(),
        scratch_types=[
            pltpu.VMEM((RB, K, D), jnp.float32),
            pltpu.VMEM((RB, K), jnp.int32),
            pltpu.VMEM((RB, K), jnp.int32),
            pltpu.VMEM_SHARED((NP, D), jnp.float32),
            pltpu.SemaphoreType.DMA((RB,)),
            pltpu.SemaphoreType.DMA((RB,)),
            pltpu.SemaphoreType.DMA((RB,)),
            pltpu.SemaphoreType.DMA((RB,)),
        ],
    )


_RB = 1000


def _mmscale_body(x_ref, w_ref, da_ref, db_ref, y_ref):
    deg = da_ref[0, :, 0:1] + db_ref[0, :, 0:1] + 1.0
    xw = jnp.dot(x_ref[...], w_ref[...], preferred_element_type=jnp.float32)
    y_ref[...] = xw * lax.rsqrt(deg)


def _tc_mmscale(x, W, degp3):
    return pl.pallas_call(
        _mmscale_body,
        grid=(N // _RB,),
        in_specs=[
            pl.BlockSpec((_RB, D), lambda i: (i, 0)),
            pl.BlockSpec((D, D), lambda i: (0, 0)),
            pl.BlockSpec((1, _RB, DW), lambda i: (0, i, 0)),
            pl.BlockSpec((1, _RB, DW), lambda i: (1, i, 0)),
        ],
        out_specs=pl.BlockSpec((_RB, D), lambda i: (i, 0)),
        out_shape=jax.ShapeDtypeStruct((N, D), jnp.float32),
    )(x, W, degp3, degp3)


def _final_body(aa_ref, ab_ref, y_ref, da_ref, db_ref, b_ref, o_ref):
    deg = da_ref[0, :, 0:1] + db_ref[0, :, 0:1] + 1.0
    dinv = lax.rsqrt(deg)
    acc = aa_ref[0] + ab_ref[0] + y_ref[...]
    o_ref[...] = jnp.maximum(dinv * acc + b_ref[...], 0.0)


def _tc_final(aggp3, y, degp3, b):
    return pl.pallas_call(
        _final_body,
        grid=(N // _RB,),
        in_specs=[
            pl.BlockSpec((1, _RB, D), lambda i: (0, i, 0)),
            pl.BlockSpec((1, _RB, D), lambda i: (1, i, 0)),
            pl.BlockSpec((_RB, D), lambda i: (i, 0)),
            pl.BlockSpec((1, _RB, DW), lambda i: (0, i, 0)),
            pl.BlockSpec((1, _RB, DW), lambda i: (1, i, 0)),
            pl.BlockSpec((1, D), lambda i: (0, 0)),
        ],
        out_specs=pl.BlockSpec((_RB, D), lambda i: (i, 0)),
        out_shape=jax.ShapeDtypeStruct((N, D), jnp.float32),
    )(aggp3, aggp3, y, degp3, degp3, b)


def kernel(x, edge_index, W, b):
    src = edge_index[0].astype(jnp.int32)
    dst = edge_index[1].astype(jnp.int32)

    degp = _sc_deg()(dst)
    degp3 = degp.reshape(NC, NP, DW)
    y = _tc_mmscale(x, W, degp3)
    aggp = _sc_agg()(y, src, dst)
    aggp3 = aggp.reshape(NC, NP, D)
    return _tc_final(aggp3, y, degp3, b.reshape(1, D).astype(jnp.float32))


# speedup vs baseline: 42.9853x; 1.0010x over previous
import functools

import jax
import jax.numpy as jnp
from jax import lax
from jax.experimental import pallas as pl
from jax.experimental.pallas import tpu as pltpu
from jax.experimental.pallas import tpu_sc as plsc

N = 10000
E = 320000
D = 128

NC = 2
NS = 16
NW = NC * NS
K = 80
NCHUNK = 125
EPT = E // NW
NP = 10112
RPT = NP // NS
DW = 16

@functools.lru_cache(maxsize=None)
def _mesh():
    return plsc.VectorSubcoreMesh(core_axis_name="c", subcore_axis_name="s",
                                  num_cores=NC, num_subcores=NS)


def _zero_rows(ref, nrows, ncol16):
    def body(i, _):
        for j in range(ncol16):
            ref[i, pl.ds(j * 16, 16)] = jnp.zeros((16,), jnp.float32)
        return 0
    lax.fori_loop(0, nrows, body, 0)


RI = 5
RD = 5
RW = 4
RB = 4


_SPANS = [(j * K, K) for j in range(RPT // K)] + (
    [(RPT - RPT % K, RPT % K)] if RPT % K else [])


def _fill_span(buf, dst, d_base):
    for off, sz in _SPANS:
        pltpu.sync_copy(buf.at[pl.ds(0, sz)],
                        dst.at[pl.ds(d_base + off, sz)])


def _writeout_sync(shared, buf, out_hbm, r0, o0):
    for off, sz in _SPANS:
        pltpu.sync_copy(shared.at[pl.ds(r0 + off, sz)],
                        buf.at[pl.ds(0, sz)])
        pltpu.sync_copy(buf.at[pl.ds(0, sz)],
                        out_hbm.at[pl.ds(o0 + off, sz)])


def _deg_body(dst_hbm, out_hbm, buf_v, idx_v, deg_sh, sem_i, sem_s):
    c = lax.axis_index("c")
    s = lax.axis_index("s")
    t = s * NC + c
    r0 = s * RPT

    def ifire(ci, u):
        base = pl.multiple_of(t * EPT + ci * K, 8)
        pltpu.async_copy(dst_hbm.at[pl.ds(base, K)], idx_v.at[ci],
                         sem_i.at[u])

    def idrain(u):
        pltpu.make_async_copy(dst_hbm.at[pl.ds(0, K)], idx_v.at[0],
                              sem_i.at[u]).wait()

    _zero_rows(buf_v, K, DW // 16)
    _fill_span(buf_v, deg_sh, r0)

    def ones_row(i, _):
        buf_v[i, :] = jnp.ones((DW,), jnp.float32)
        return 0
    lax.fori_loop(0, K, ones_row, 0)

    for u in range(RI):
        ifire(u, u)

    def istep(gi, _):
        for u in range(RI):
            idrain(u)
            ifire(gi * RI + u, u)
        return 0
    lax.fori_loop(1, NCHUNK // RI, istep, 0)
    for u in range(RI):
        idrain(u)
    plsc.subcore_barrier()

    def start_s(ci, u):
        pltpu.async_copy(buf_v, deg_sh.at[idx_v.at[ci]], sem_s.at[u],
                         add=True)

    def wait_s(u):
        pltpu.make_async_copy(buf_v, deg_sh.at[idx_v.at[0]],
                              sem_s.at[u]).wait()

    for u in range(RD):
        start_s(u, u)

    def step(gi, _):
        for u in range(RD):
            wait_s(u)
            start_s(gi * RD + u, u)
        return 0
    lax.fori_loop(1, NCHUNK // RD, step, 0)
    for u in range(RD):
        wait_s(u)
    plsc.subcore_barrier()

    _writeout_sync(deg_sh, buf_v, out_hbm, r0, c * NP + r0)


@functools.lru_cache(maxsize=None)
def _sc_deg():
    return pl.kernel(
        _deg_body,
        out_type=jax.ShapeDtypeStruct((NC * NP, DW), jnp.float32),
        mesh=_mesh(),
        scratch_types=[
            pltpu.VMEM((K, DW), jnp.float32),
            pltpu.VMEM((NCHUNK, K), jnp.int32),
            pltpu.VMEM_SHARED((NP, DW), jnp.float32),
            pltpu.SemaphoreType.DMA((RI,)),
            pltpu.SemaphoreType.DMA((RD,)),
        ],
    )


def _agg_body(y_hbm, src_hbm, dst_hbm, out_hbm, bufs_v, srcr_v, dstr_v,
              acc_sh, sem_is, sem_id, sem_g, sem_s):
    c = lax.axis_index("c")
    s = lax.axis_index("s")
    t = s * NC + c
    r0 = s * RPT
    _zero_rows(bufs_v.at[0], K, D // 16)
    _fill_span(bufs_v.at[0], acc_sh, r0)
    plsc.subcore_barrier()

    def start_il(ci, u):
        base = pl.multiple_of(t * EPT, 8) + ci * K
        pltpu.async_copy(src_hbm.at[pl.ds(base, K)], srcr_v.at[u],
                         sem_is.at[u])
        pltpu.async_copy(dst_hbm.at[pl.ds(base, K)], dstr_v.at[u],
                         sem_id.at[u])

    def wait_il(u):
        pltpu.make_async_copy(src_hbm.at[pl.ds(0, K)], srcr_v.at[u],
                              sem_is.at[u]).wait()
        pltpu.make_async_copy(dst_hbm.at[pl.ds(0, K)], dstr_v.at[u],
                              sem_id.at[u]).wait()

    def start_g(u):
        pltpu.async_copy(y_hbm.at[srcr_v.at[u]], bufs_v.at[u], sem_g.at[u])

    def wait_g(u):
        pltpu.make_async_copy(y_hbm.at[srcr_v.at[u]], bufs_v.at[u],
                              sem_g.at[u]).wait()

    def start_s(u):
        pltpu.async_copy(bufs_v.at[u], acc_sh.at[dstr_v.at[u]],
                         sem_s.at[u], add=True)

    def wait_s(u):
        pltpu.make_async_copy(bufs_v.at[u], acc_sh.at[dstr_v.at[u]],
                              sem_s.at[u]).wait()

    LAST = NCHUNK - 1
    start_il(0, 0)
    start_il(1, 1)
    start_il(2, 2)
    wait_il(0)
    start_g(0)
    wait_il(1)
    start_g(1)
    for i in range(RB):
        if i >= 1:
            wait_s(i - 1)
        start_il(i + 3, (i + 3) % RB)
        wait_il((i + 2) % RB)
        start_g((i + 2) % RB)
        wait_g(i)
        start_s(i)

    def step(gi, _):
        for u in range(RB):
            i = gi * RB + u
            wait_s((u + 3) % RB)
            start_il(jnp.minimum(i + 3, LAST), (u + 3) % RB)
            wait_il((u + 2) % RB)
            start_g((u + 2) % RB)
            wait_g(u)
            start_s(u)
        return 0
    lax.fori_loop(1, NCHUNK // RB, step, 0)

    wait_s(3)
    wait_g(0)
    start_s(0)
    wait_s(0)
    wait_g(1)
    wait_il(2)
    plsc.subcore_barrier()

    _writeout_sync(acc_sh, bufs_v.at[0], out_hbm, r0, c * NP + r0)


@functools.lru_cache(maxsize=None)
def _sc_agg():
    return pl.kernel(
        _agg_body,
        out_type=jax.ShapeDtypeStruct((NC * NP, D), jnp.float32),
        mesh=_mesh(),
        scratch_types=[
            pltpu.VMEM((RB, K, D), jnp.float32),
            pltpu.VMEM((RB, K), jnp.int32),
            pltpu.VMEM((RB, K), jnp.int32),
            pltpu.VMEM_SHARED((NP, D), jnp.float32),
            pltpu.SemaphoreType.DMA((RB,)),
            pltpu.SemaphoreType.DMA((RB,)),
            pltpu.SemaphoreType.DMA((RB,)),
            pltpu.SemaphoreType.DMA((RB,)),
        ],
    )


_RB = 1000


def _mmscale_body(x_ref, w_ref, da_ref, db_ref, y_ref):
    deg = da_ref[0, :, 0:1] + db_ref[0, :, 0:1] + 1.0
    xw = jnp.dot(x_ref[...], w_ref[...], preferred_element_type=jnp.float32)
    y_ref[...] = xw * lax.rsqrt(deg)


def _tc_mmscale(x, W, degp3):
    return pl.pallas_call(
        _mmscale_body,
        grid=(N // _RB,),
        in_specs=[
            pl.BlockSpec((_RB, D), lambda i: (i, 0)),
            pl.BlockSpec((D, D), lambda i: (0, 0)),
            pl.BlockSpec((1, _RB, DW), lambda i: (0, i, 0)),
            pl.BlockSpec((1, _RB, DW), lambda i: (1, i, 0)),
        ],
        out_specs=pl.BlockSpec((_RB, D), lambda i: (i, 0)),
        out_shape=jax.ShapeDtypeStruct((N, D), jnp.float32),
    )(x, W, degp3, degp3)


def _final_body(aa_ref, ab_ref, y_ref, da_ref, db_ref, b_ref, o_ref):
    deg = da_ref[0, :, 0:1] + db_ref[0, :, 0:1] + 1.0
    dinv = lax.rsqrt(deg)
    acc = aa_ref[0] + ab_ref[0] + y_ref[...]
    o_ref[...] = jnp.maximum(dinv * acc + b_ref[...], 0.0)


def _tc_final(aggp3, y, degp3, b):
    return pl.pallas_call(
        _final_body,
        grid=(N // _RB,),
        in_specs=[
            pl.BlockSpec((1, _RB, D), lambda i: (0, i, 0)),
            pl.BlockSpec((1, _RB, D), lambda i: (1, i, 0)),
            pl.BlockSpec((_RB, D), lambda i: (i, 0)),
            pl.BlockSpec((1, _RB, DW), lambda i: (0, i, 0)),
            pl.BlockSpec((1, _RB, DW), lambda i: (1, i, 0)),
            pl.BlockSpec((1, D), lambda i: (0, 0)),
        ],
        out_specs=pl.BlockSpec((_RB, D), lambda i: (i, 0)),
        out_shape=jax.ShapeDtypeStruct((N, D), jnp.float32),
    )(aggp3, aggp3, y, degp3, degp3, b)


def kernel(x, edge_index, W, b):
    src = edge_index[0].astype(jnp.int32)
    dst = edge_index[1].astype(jnp.int32)

    degp = _sc_deg()(dst)
    degp3 = degp.reshape(NC, NP, DW)
    y = _tc_mmscale(x, W, degp3)
    aggp = _sc_agg()(y, src, dst)
    aggp3 = aggp.reshape(NC, NP, D)
    return _tc_final(aggp3, y, degp3, b.reshape(1, D).astype(jnp.float32))
